# baseline (speedup 1.0000x reference)
import math

import jax
import jax.numpy as jnp
from jax import lax
from jax.experimental import pallas as pl
from jax.experimental.pallas import tpu as pltpu
from jax.experimental.pallas import tpu_sc as plsc

VOCAB_SIZE = 1000000
D_MODEL = 64
BATCH = 4096
SEQ_LEN = 200
SCALE = math.sqrt(D_MODEL)

NC = 2
NS = 16
NW = NC * NS
N_TOK = BATCH * SEQ_LEN
B_PER_W = N_TOK // NW
BLK = 128
N_BLK = B_PER_W // BLK
BT = BATCH // BLK
TROW = 8 * BLK


def _emb_kernel(x_hbm, lut_hbm, out_hbm, idx_all,
                rows0, rows1, ot0, ot1,
                sem_g0, sem_g1, sem_o0, sem_o1):
    wid = lax.axis_index("s") * NC + lax.axis_index("c")
    base = wid * B_PER_W
    rows = (rows0, rows1)
    ot = (ot0, ot1)
    sem_g = (sem_g0, sem_g1)
    sem_o = (sem_o0, sem_o1)

    pltpu.sync_copy(x_hbm.at[pl.ds(base, B_PER_W)], idx_all)

    @pl.loop(0, B_PER_W // 16, unroll=8)
    def remap_loop(i):
        sl = pl.ds(i * 16, 16)
        idx_all[sl] = idx_all[sl] + idx_all[sl]

    def gather_desc(k, b):
        return pltpu.make_async_copy(
            lut_hbm.at[idx_all.at[pl.ds(k * BLK, BLK)]], rows[b], sem_g[b])

    def wb_descs(k, b):
        t0 = base + k * BLK
        s = t0 // BATCH
        tc = (t0 % BATCH) // BLK
        return [
            pltpu.make_async_copy(
                ot[b].at[tr, :, pl.ds(0, BLK)],
                out_hbm.at[s, tr, tc, :, :], sem_o[b])
            for tr in range(D_MODEL // 8)
        ]

    iota = lax.iota(jnp.int32, 16)

    gather_desc(0, 0).start()
    gather_desc(1, 1).start()

    @pl.loop(0, N_BLK, step=2)
    def blk_loop(k0):
        for b in range(2):
            k = k0 + b
            gather_desc(k, b).wait()

            @pl.when(k >= 2)
            def _():
                for d in wb_descs(k - 2, b):
                    d.wait()

            for d16 in range(D_MODEL // 16):
                dv = iota + (d16 * 16)
                tr_v = lax.shift_right_logical(dv, 3)
                r_v = lax.bitwise_and(dv, 7)

                @plsc.parallel_loop(0, BLK, unroll=8)
                def tok_loop(j):
                    jv = jnp.zeros((16,), jnp.int32) + j
                    v = rows[b][j, pl.ds(d16 * 16, 16)]
                    plsc.store_scatter(ot[b], [tr_v, r_v, jv], v * SCALE)

            @pl.when(k + 2 < N_BLK)
            def _():
                gather_desc(k + 2, b).start()

            for d in wb_descs(k, b):
                d.start()

    for b in range(2):
        for d in wb_descs(N_BLK - 2 + b, b):
            d.wait()


VOCAB_PAD = 2 * VOCAB_SIZE


@jax.jit
def kernel(x, lut):
    x_flat = jnp.swapaxes(x, 0, 1).reshape(-1).astype(jnp.int32)
    lut = jnp.pad(lut, ((0, 0), (0, 64))).reshape(VOCAB_PAD, D_MODEL)
    mesh = plsc.VectorSubcoreMesh(core_axis_name="c", subcore_axis_name="s")
    out4 = pl.kernel(
        _emb_kernel,
        out_type=jax.ShapeDtypeStruct(
            (SEQ_LEN, D_MODEL // 8, BT, 8, BLK), jnp.float32),
        mesh=mesh,
        scratch_types=[
            pltpu.VMEM((B_PER_W,), jnp.int32),
            pltpu.VMEM((BLK, D_MODEL), jnp.float32),
            pltpu.VMEM((BLK, D_MODEL), jnp.float32),
            pltpu.VMEM((D_MODEL // 8, 8, BLK + 1), jnp.float32),
            pltpu.VMEM((D_MODEL // 8, 8, BLK + 1), jnp.float32),
            pltpu.SemaphoreType.DMA,
            pltpu.SemaphoreType.DMA,
            pltpu.SemaphoreType.DMA,
            pltpu.SemaphoreType.DMA,
        ],
        compiler_params=pltpu.CompilerParams(
            use_tc_tiling_on_sc=False, needs_layout_passes=False),
    )(x_flat, lut)
    return out4.transpose(2, 4, 0, 1, 3).reshape(BATCH, SEQ_LEN, D_MODEL)


# speedup vs baseline: 1.4010x; 1.4010x over previous
import math

import jax
import jax.numpy as jnp
from jax import lax
from jax.experimental import pallas as pl
from jax.experimental.pallas import tpu as pltpu
from jax.experimental.pallas import tpu_sc as plsc

VOCAB_SIZE = 1000000
D_MODEL = 64
BATCH = 4096
SEQ_LEN = 200
SCALE = math.sqrt(D_MODEL)

NC = 2
NS = 16
NW = NC * NS
N_TOK = BATCH * SEQ_LEN
B_PER_W = N_TOK // NW
BLK = 128
N_BLK = B_PER_W // BLK
BT = BATCH // BLK
TROW = 8 * BLK


def _emb_kernel(x_hbm, lut_hbm, out_hbm, idx_all,
                rows0, rows1, ot0, ot1,
                sem_g0, sem_g1, sem_o0, sem_o1):
    wid = lax.axis_index("s") * NC + lax.axis_index("c")
    base = wid * B_PER_W
    rows = (rows0, rows1)
    ot = (ot0, ot1)
    sem_g = (sem_g0, sem_g1)
    sem_o = (sem_o0, sem_o1)

    pltpu.sync_copy(x_hbm.at[pl.ds(base, B_PER_W)], idx_all)

    @pl.loop(0, B_PER_W // 16, unroll=8)
    def remap_loop(i):
        sl = pl.ds(i * 16, 16)
        v = idx_all[sl]
        hi = lax.bitwise_and(v, -4096)
        lo2 = lax.shift_left(lax.bitwise_and(v, 2047), 1)
        h = lax.bitwise_and(lax.shift_right_logical(v, 11), 1)
        idx_all[sl] = hi + lo2 + h

    def gather_desc(k, b):
        return pltpu.make_async_copy(
            lut_hbm.at[idx_all.at[pl.ds(k * BLK, BLK)]], rows[b], sem_g[b])

    def wb_descs(k, b):
        t0 = base + k * BLK
        s = t0 // BATCH
        tc = (t0 % BATCH) // BLK
        return [
            pltpu.make_async_copy(
                ot[b].at[tr, :, pl.ds(0, BLK)],
                out_hbm.at[s, tr, tc, :, :], sem_o[b])
            for tr in range(D_MODEL // 8)
        ]

    iota = lax.iota(jnp.int32, 16)

    gather_desc(0, 0).start()
    gather_desc(1, 1).start()

    @pl.loop(0, N_BLK, step=2)
    def blk_loop(k0):
        for b in range(2):
            k = k0 + b
            gather_desc(k, b).wait()

            @pl.when(k >= 2)
            def _():
                for d in wb_descs(k - 2, b):
                    d.wait()

            for d16 in range(D_MODEL // 16):
                dv = iota + (d16 * 16)
                tr_v = lax.shift_right_logical(dv, 3)
                r_v = lax.bitwise_and(dv, 7)

                @plsc.parallel_loop(0, BLK, unroll=8)
                def tok_loop(j):
                    jv = jnp.zeros((16,), jnp.int32) + j
                    v = rows[b][j, pl.ds(d16 * 16, 16)]
                    plsc.store_scatter(ot[b], [tr_v, r_v, jv], v)

            @pl.when(k + 2 < N_BLK)
            def _():
                gather_desc(k + 2, b).start()

            for d in wb_descs(k, b):
                d.start()

    for b in range(2):
        for d in wb_descs(N_BLK - 2 + b, b):
            d.wait()


def _retile_kernel(a_ref, b_ref, o_ref):
    o_ref[:, 0:D_MODEL] = a_ref[...].T * SCALE
    o_ref[:, D_MODEL:2 * D_MODEL] = b_ref[...].T * SCALE


_RETILE_W = 2048
_RETILE_G = -(-VOCAB_SIZE // (2 * _RETILE_W))
VOCAB_PAD = _RETILE_G * 2 * _RETILE_W
_LAST_IN_BLK = (VOCAB_SIZE - 1) // _RETILE_W


def _retile(lut_t):
    return pl.pallas_call(
        _retile_kernel,
        grid=(_RETILE_G,),
        in_specs=[
            pl.BlockSpec((D_MODEL, _RETILE_W), lambda k: (0, 2 * k)),
            pl.BlockSpec(
                (D_MODEL, _RETILE_W),
                lambda k: (0, jnp.minimum(2 * k + 1, _LAST_IN_BLK))),
        ],
        out_specs=pl.BlockSpec((_RETILE_W, 2 * D_MODEL), lambda k: (k, 0)),
        out_shape=jax.ShapeDtypeStruct(
            (VOCAB_PAD // 2, 2 * D_MODEL), jnp.float32),
    )(lut_t, lut_t)


@jax.jit
def kernel(x, lut):
    x_flat = jnp.swapaxes(x, 0, 1).reshape(-1).astype(jnp.int32)
    lut = _retile(lut.T).reshape(VOCAB_PAD, D_MODEL)
    mesh = plsc.VectorSubcoreMesh(core_axis_name="c", subcore_axis_name="s")
    out4 = pl.kernel(
        _emb_kernel,
        out_type=jax.ShapeDtypeStruct(
            (SEQ_LEN, D_MODEL // 8, BT, 8, BLK), jnp.float32),
        mesh=mesh,
        scratch_types=[
            pltpu.VMEM((B_PER_W,), jnp.int32),
            pltpu.VMEM((BLK, D_MODEL), jnp.float32),
            pltpu.VMEM((BLK, D_MODEL), jnp.float32),
            pltpu.VMEM((D_MODEL // 8, 8, BLK + 1), jnp.float32),
            pltpu.VMEM((D_MODEL // 8, 8, BLK + 1), jnp.float32),
            pltpu.SemaphoreType.DMA,
            pltpu.SemaphoreType.DMA,
            pltpu.SemaphoreType.DMA,
            pltpu.SemaphoreType.DMA,
        ],
        compiler_params=pltpu.CompilerParams(
            use_tc_tiling_on_sc=False, needs_layout_passes=False),
    )(x_flat, lut)
    return out4.transpose(2, 4, 0, 1, 3).reshape(BATCH, SEQ_LEN, D_MODEL)


# speedup vs baseline: 1.6051x; 1.1457x over previous
import math

import jax
import jax.numpy as jnp
from jax import lax
from jax.experimental import pallas as pl
from jax.experimental.pallas import tpu as pltpu
from jax.experimental.pallas import tpu_sc as plsc

VOCAB_SIZE = 1000000
D_MODEL = 64
BATCH = 4096
SEQ_LEN = 200
SCALE = math.sqrt(D_MODEL)

NC = 2
NS = 16
NW = NC * NS
N_TOK = BATCH * SEQ_LEN
B_PER_W = N_TOK // NW
BLK = 128
N_BLK = B_PER_W // BLK
BT = BATCH // BLK
TROW = 8 * BLK


def _emb_kernel(x_hbm, lut_hbm, out_hbm, idx_all,
                rows0, rows1, ot0, ot1,
                sem_g0, sem_g1, sem_o0, sem_o1):
    wid = lax.axis_index("s") * NC + lax.axis_index("c")
    base = wid * B_PER_W
    rows = (rows0, rows1)
    ot = (ot0, ot1)
    sem_g = (sem_g0, sem_g1)
    sem_o = (sem_o0, sem_o1)

    pltpu.sync_copy(x_hbm.at[pl.ds(base, B_PER_W)], idx_all)

    @pl.loop(0, B_PER_W // 16, unroll=8)
    def remap_loop(i):
        sl = pl.ds(i * 16, 16)
        v = idx_all[sl]
        hi = lax.bitwise_and(v, -8192)
        lo2 = lax.shift_left(lax.bitwise_and(v, 4095), 1)
        h = lax.bitwise_and(lax.shift_right_logical(v, 12), 1)
        idx_all[sl] = hi + lo2 + h

    def gather_desc(k, b):
        return pltpu.make_async_copy(
            lut_hbm.at[idx_all.at[pl.ds(k * BLK, BLK)]], rows[b], sem_g[b])

    def wb_descs(k, b):
        t0 = base + k * BLK
        s = t0 // BATCH
        tc = (t0 % BATCH) // BLK
        return [
            pltpu.make_async_copy(
                ot[b].at[tr, :, pl.ds(0, BLK)],
                out_hbm.at[s, tr, tc, :, :], sem_o[b])
            for tr in range(D_MODEL // 8)
        ]

    iota = lax.iota(jnp.int32, 16)

    gather_desc(0, 0).start()
    gather_desc(1, 1).start()

    @pl.loop(0, N_BLK, step=2)
    def blk_loop(k0):
        for b in range(2):
            k = k0 + b
            gather_desc(k, b).wait()

            @pl.when(k >= 2)
            def _():
                for d in wb_descs(k - 2, b):
                    d.wait()

            for d16 in range(D_MODEL // 16):
                dv = iota + (d16 * 16)
                tr_v = lax.shift_right_logical(dv, 3)
                r_v = lax.bitwise_and(dv, 7)

                @plsc.parallel_loop(0, BLK, unroll=8)
                def tok_loop(j):
                    jv = jnp.zeros((16,), jnp.int32) + j
                    v = rows[b][j, pl.ds(d16 * 16, 16)]
                    plsc.store_scatter(ot[b], [tr_v, r_v, jv], v)

            @pl.when(k + 2 < N_BLK)
            def _():
                gather_desc(k + 2, b).start()

            for d in wb_descs(k, b):
                d.start()

    for b in range(2):
        for d in wb_descs(N_BLK - 2 + b, b):
            d.wait()


def _retile_kernel(a_ref, b_ref, o_ref):
    o_ref[:, 0:D_MODEL] = a_ref[...].T * SCALE
    o_ref[:, D_MODEL:2 * D_MODEL] = b_ref[...].T * SCALE


_RETILE_W = 4096
_RETILE_G = -(-VOCAB_SIZE // (2 * _RETILE_W))
VOCAB_PAD = _RETILE_G * 2 * _RETILE_W
_LAST_IN_BLK = (VOCAB_SIZE - 1) // _RETILE_W


def _retile(lut_t):
    return pl.pallas_call(
        _retile_kernel,
        grid=(_RETILE_G,),
        in_specs=[
            pl.BlockSpec((D_MODEL, _RETILE_W), lambda k: (0, 2 * k)),
            pl.BlockSpec(
                (D_MODEL, _RETILE_W),
                lambda k: (0, jnp.minimum(2 * k + 1, _LAST_IN_BLK))),
        ],
        out_specs=pl.BlockSpec((_RETILE_W, 2 * D_MODEL), lambda k: (k, 0)),
        out_shape=jax.ShapeDtypeStruct(
            (VOCAB_PAD // 2, 2 * D_MODEL), jnp.float32),
    )(lut_t, lut_t)


@jax.jit
def kernel(x, lut):
    x_flat = jnp.swapaxes(x, 0, 1).reshape(-1).astype(jnp.int32)
    lut = _retile(lut.T).reshape(VOCAB_PAD, D_MODEL)
    mesh = plsc.VectorSubcoreMesh(core_axis_name="c", subcore_axis_name="s")
    out4 = pl.kernel(
        _emb_kernel,
        out_type=jax.ShapeDtypeStruct(
            (SEQ_LEN, D_MODEL // 8, BT, 8, BLK), jnp.float32),
        mesh=mesh,
        scratch_types=[
            pltpu.VMEM((B_PER_W,), jnp.int32),
            pltpu.VMEM((BLK, D_MODEL), jnp.float32),
            pltpu.VMEM((BLK, D_MODEL), jnp.float32),
            pltpu.VMEM((D_MODEL // 8, 8, BLK + 1), jnp.float32),
            pltpu.VMEM((D_MODEL // 8, 8, BLK + 1), jnp.float32),
            pltpu.SemaphoreType.DMA,
            pltpu.SemaphoreType.DMA,
            pltpu.SemaphoreType.DMA,
            pltpu.SemaphoreType.DMA,
        ],
        compiler_params=pltpu.CompilerParams(
            use_tc_tiling_on_sc=False, needs_layout_passes=False),
    )(x_flat, lut)
    return out4.transpose(2, 4, 0, 1, 3).reshape(BATCH, SEQ_LEN, D_MODEL)


# speedup vs baseline: 1.7271x; 1.0760x over previous
import math

import jax
import jax.numpy as jnp
from jax import lax
from jax.experimental import pallas as pl
from jax.experimental.pallas import tpu as pltpu
from jax.experimental.pallas import tpu_sc as plsc

VOCAB_SIZE = 1000000
D_MODEL = 64
BATCH = 4096
SEQ_LEN = 200
SCALE = math.sqrt(D_MODEL)

NC = 2
NS = 16
NW = NC * NS
N_TOK = BATCH * SEQ_LEN
B_PER_W = N_TOK // NW
BLK = 128
N_BLK = B_PER_W // BLK
BT = BATCH // BLK
TROW = 8 * BLK


def _emb_kernel(x_hbm, lut_hbm, out_hbm, idx_all,
                rows0, rows1, ot0, ot1,
                sem_g0, sem_g1, sem_o0, sem_o1):
    wid = lax.axis_index("s") * NC + lax.axis_index("c")
    base = wid * B_PER_W
    rows = (rows0, rows1)
    ot = (ot0, ot1)
    sem_g = (sem_g0, sem_g1)
    sem_o = (sem_o0, sem_o1)

    pltpu.sync_copy(x_hbm.at[pl.ds(base, B_PER_W)], idx_all)

    @pl.loop(0, B_PER_W // 16, unroll=8)
    def remap_loop(i):
        sl = pl.ds(i * 16, 16)
        v = idx_all[sl]
        hi = lax.bitwise_and(v, -16384)
        lo2 = lax.shift_left(lax.bitwise_and(v, 8191), 1)
        h = lax.bitwise_and(lax.shift_right_logical(v, 13), 1)
        idx_all[sl] = hi + lo2 + h

    def gather_desc(k, b):
        return pltpu.make_async_copy(
            lut_hbm.at[idx_all.at[pl.ds(k * BLK, BLK)]], rows[b], sem_g[b])

    def wb_descs(k, b):
        t0 = base + k * BLK
        s = t0 // BATCH
        tc = (t0 % BATCH) // BLK
        return [
            pltpu.make_async_copy(
                ot[b].at[tr, :, pl.ds(0, BLK)],
                out_hbm.at[s, tr, tc, :, :], sem_o[b])
            for tr in range(D_MODEL // 8)
        ]

    iota = lax.iota(jnp.int32, 16)

    gather_desc(0, 0).start()
    gather_desc(1, 1).start()

    @pl.loop(0, N_BLK, step=2)
    def blk_loop(k0):
        for b in range(2):
            k = k0 + b
            gather_desc(k, b).wait()

            @pl.when(k >= 2)
            def _():
                for d in wb_descs(k - 2, b):
                    d.wait()

            for d16 in range(D_MODEL // 16):
                dv = iota + (d16 * 16)
                tr_v = lax.shift_right_logical(dv, 3)
                r_v = lax.bitwise_and(dv, 7)

                @plsc.parallel_loop(0, BLK, unroll=8)
                def tok_loop(j):
                    jv = jnp.zeros((16,), jnp.int32) + j
                    v = rows[b][j, pl.ds(d16 * 16, 16)]
                    plsc.store_scatter(ot[b], [tr_v, r_v, jv], v)

            @pl.when(k + 2 < N_BLK)
            def _():
                gather_desc(k + 2, b).start()

            for d in wb_descs(k, b):
                d.start()

    for b in range(2):
        for d in wb_descs(N_BLK - 2 + b, b):
            d.wait()


def _retile_kernel(a_ref, b_ref, o_ref):
    o_ref[:, 0:D_MODEL] = a_ref[...].T * SCALE
    o_ref[:, D_MODEL:2 * D_MODEL] = b_ref[...].T * SCALE


_RETILE_W = 8192
_RETILE_G = -(-VOCAB_SIZE // (2 * _RETILE_W))
VOCAB_PAD = _RETILE_G * 2 * _RETILE_W
_LAST_IN_BLK = (VOCAB_SIZE - 1) // _RETILE_W


def _retile(lut_t):
    return pl.pallas_call(
        _retile_kernel,
        grid=(_RETILE_G,),
        in_specs=[
            pl.BlockSpec((D_MODEL, _RETILE_W), lambda k: (0, 2 * k)),
            pl.BlockSpec(
                (D_MODEL, _RETILE_W),
                lambda k: (0, jnp.minimum(2 * k + 1, _LAST_IN_BLK))),
        ],
        out_specs=pl.BlockSpec((_RETILE_W, 2 * D_MODEL), lambda k: (k, 0)),
        out_shape=jax.ShapeDtypeStruct(
            (VOCAB_PAD // 2, 2 * D_MODEL), jnp.float32),
    )(lut_t, lut_t)


@jax.jit
def kernel(x, lut):
    x_flat = jnp.swapaxes(x, 0, 1).reshape(-1).astype(jnp.int32)
    lut = _retile(lut.T).reshape(VOCAB_PAD, D_MODEL)
    mesh = plsc.VectorSubcoreMesh(core_axis_name="c", subcore_axis_name="s")
    out4 = pl.kernel(
        _emb_kernel,
        out_type=jax.ShapeDtypeStruct(
            (SEQ_LEN, D_MODEL // 8, BT, 8, BLK), jnp.float32),
        mesh=mesh,
        scratch_types=[
            pltpu.VMEM((B_PER_W,), jnp.int32),
            pltpu.VMEM((BLK, D_MODEL), jnp.float32),
            pltpu.VMEM((BLK, D_MODEL), jnp.float32),
            pltpu.VMEM((D_MODEL // 8, 8, BLK + 1), jnp.float32),
            pltpu.VMEM((D_MODEL // 8, 8, BLK + 1), jnp.float32),
            pltpu.SemaphoreType.DMA,
            pltpu.SemaphoreType.DMA,
            pltpu.SemaphoreType.DMA,
            pltpu.SemaphoreType.DMA,
        ],
        compiler_params=pltpu.CompilerParams(
            use_tc_tiling_on_sc=False, needs_layout_passes=False),
    )(x_flat, lut)
    return out4.transpose(2, 4, 0, 1, 3).reshape(BATCH, SEQ_LEN, D_MODEL)


# speedup vs baseline: 1.7856x; 1.0339x over previous
import math

import jax
import jax.numpy as jnp
from jax import lax
from jax.experimental import pallas as pl
from jax.experimental.pallas import tpu as pltpu
from jax.experimental.pallas import tpu_sc as plsc

VOCAB_SIZE = 1000000
D_MODEL = 64
BATCH = 4096
SEQ_LEN = 200
SCALE = math.sqrt(D_MODEL)

NC = 2
NS = 16
NW = NC * NS
N_TOK = BATCH * SEQ_LEN
B_PER_W = N_TOK // NW
BLK = 128
N_BLK = B_PER_W // BLK
BT = BATCH // BLK
TROW = 8 * BLK


def _emb_kernel(x_hbm, lut_hbm, out_hbm, idx_all,
                rows0, rows1, ot0, ot1,
                sem_g0, sem_g1, sem_o0, sem_o1):
    wid = lax.axis_index("s") * NC + lax.axis_index("c")
    base = wid * B_PER_W
    rows = (rows0, rows1)
    ot = (ot0, ot1)
    sem_g = (sem_g0, sem_g1)
    sem_o = (sem_o0, sem_o1)

    pltpu.sync_copy(x_hbm.at[pl.ds(base, B_PER_W)], idx_all)

    @pl.loop(0, B_PER_W // 16, unroll=8)
    def remap_loop(i):
        sl = pl.ds(i * 16, 16)
        v = idx_all[sl]
        hi = lax.bitwise_and(v, -32768)
        lo2 = lax.shift_left(lax.bitwise_and(v, 16383), 1)
        h = lax.bitwise_and(lax.shift_right_logical(v, 14), 1)
        idx_all[sl] = hi + lo2 + h

    def gather_desc(k, b):
        return pltpu.make_async_copy(
            lut_hbm.at[idx_all.at[pl.ds(k * BLK, BLK)]], rows[b], sem_g[b])

    def wb_descs(k, b):
        t0 = base + k * BLK
        s = t0 // BATCH
        tc = (t0 % BATCH) // BLK
        return [
            pltpu.make_async_copy(
                ot[b].at[tr, :, pl.ds(0, BLK)],
                out_hbm.at[s, tr, tc, :, :], sem_o[b])
            for tr in range(D_MODEL // 8)
        ]

    iota = lax.iota(jnp.int32, 16)

    gather_desc(0, 0).start()
    gather_desc(1, 1).start()

    @pl.loop(0, N_BLK, step=2)
    def blk_loop(k0):
        for b in range(2):
            k = k0 + b
            gather_desc(k, b).wait()

            @pl.when(k >= 2)
            def _():
                for d in wb_descs(k - 2, b):
                    d.wait()

            for d16 in range(D_MODEL // 16):
                dv = iota + (d16 * 16)
                tr_v = lax.shift_right_logical(dv, 3)
                r_v = lax.bitwise_and(dv, 7)

                @plsc.parallel_loop(0, BLK, unroll=8)
                def tok_loop(j):
                    jv = jnp.zeros((16,), jnp.int32) + j
                    v = rows[b][j, pl.ds(d16 * 16, 16)]
                    plsc.store_scatter(ot[b], [tr_v, r_v, jv], v)

            @pl.when(k + 2 < N_BLK)
            def _():
                gather_desc(k + 2, b).start()

            for d in wb_descs(k, b):
                d.start()

    for b in range(2):
        for d in wb_descs(N_BLK - 2 + b, b):
            d.wait()


def _retile_kernel(a_ref, b_ref, o_ref):
    o_ref[:, 0:D_MODEL] = a_ref[...].T * SCALE
    o_ref[:, D_MODEL:2 * D_MODEL] = b_ref[...].T * SCALE


_RETILE_W = 16384
_RETILE_G = -(-VOCAB_SIZE // (2 * _RETILE_W))
VOCAB_PAD = _RETILE_G * 2 * _RETILE_W
_LAST_IN_BLK = (VOCAB_SIZE - 1) // _RETILE_W


def _retile(lut_t):
    return pl.pallas_call(
        _retile_kernel,
        grid=(_RETILE_G,),
        in_specs=[
            pl.BlockSpec((D_MODEL, _RETILE_W), lambda k: (0, 2 * k)),
            pl.BlockSpec(
                (D_MODEL, _RETILE_W),
                lambda k: (0, jnp.minimum(2 * k + 1, _LAST_IN_BLK))),
        ],
        out_specs=pl.BlockSpec((_RETILE_W, 2 * D_MODEL), lambda k: (k, 0)),
        out_shape=jax.ShapeDtypeStruct(
            (VOCAB_PAD // 2, 2 * D_MODEL), jnp.float32),
    )(lut_t, lut_t)


@jax.jit
def kernel(x, lut):
    x_flat = jnp.swapaxes(x, 0, 1).reshape(-1).astype(jnp.int32)
    lut = _retile(lut.T).reshape(VOCAB_PAD, D_MODEL)
    mesh = plsc.VectorSubcoreMesh(core_axis_name="c", subcore_axis_name="s")
    out4 = pl.kernel(
        _emb_kernel,
        out_type=jax.ShapeDtypeStruct(
            (SEQ_LEN, D_MODEL // 8, BT, 8, BLK), jnp.float32),
        mesh=mesh,
        scratch_types=[
            pltpu.VMEM((B_PER_W,), jnp.int32),
            pltpu.VMEM((BLK, D_MODEL), jnp.float32),
            pltpu.VMEM((BLK, D_MODEL), jnp.float32),
            pltpu.VMEM((D_MODEL // 8, 8, BLK + 1), jnp.float32),
            pltpu.VMEM((D_MODEL // 8, 8, BLK + 1), jnp.float32),
            pltpu.SemaphoreType.DMA,
            pltpu.SemaphoreType.DMA,
            pltpu.SemaphoreType.DMA,
            pltpu.SemaphoreType.DMA,
        ],
        compiler_params=pltpu.CompilerParams(
            use_tc_tiling_on_sc=False, needs_layout_passes=False),
    )(x_flat, lut)
    return out4.transpose(2, 4, 0, 1, 3).reshape(BATCH, SEQ_LEN, D_MODEL)
